# baseline (speedup 1.0000x reference)
import functools

import jax
import jax.numpy as jnp
from jax import lax
from jax.experimental import pallas as pl
from jax.experimental.pallas import tpu as pltpu
from jax.experimental.pallas import tpu_sc as plsc

H = 16
S = 2048
NC = 2
NS = 16
NW = NC * NS
W = 15 * 128 + S
PHASES = 2
M_PER_TILE = 4


def _sc_expand_call(tabv):
    mesh = plsc.VectorSubcoreMesh(core_axis_name="c", subcore_axis_name="s")

    @functools.partial(
        pl.kernel,
        mesh=mesh,
        out_type=jax.ShapeDtypeStruct((H, S, S), jnp.float32),
        scratch_types=[
            pltpu.VMEM((8 * M_PER_TILE, W), jnp.float32),
            pltpu.SemaphoreType.DMA,
        ],
    )
    def sc_expand(tabv_hbm, out_hbm, vt, sem):
        cid = lax.axis_index("c")
        sid = lax.axis_index("s")
        w = sid * NC + cid
        hslot = w // 4
        mlo = M_PER_TILE * (w % 4)

        for phase in range(PHASES):
            h = phase * (H // PHASES) + hslot
            pltpu.sync_copy(tabv_hbm.at[h, pl.ds(8 * (12 - mlo), 32)], vt)

            def fire(a, carry):
                src = vt.at[:, pl.ds(128 * a, S)]
                dst = out_hbm.at[h, pl.ds(2016 - 8 * mlo - 128 * a, 32)]
                pltpu.make_async_copy(src, dst, sem).start()
                return carry

            lax.fori_loop(0, 16, fire, 0)

            def drain(a, carry):
                src = vt.at[:, pl.ds(0, S)]
                dst = out_hbm.at[h, pl.ds(2016 - 8 * mlo - 128 * a, 32)]
                pltpu.make_async_copy(src, dst, sem).wait()
                return carry

            lax.fori_loop(0, 16, drain, 0)

    return sc_expand(tabv)


def kernel(table, seq_len):
    del seq_len
    ftf = jnp.flip(table, axis=0).T
    tabr8 = jnp.stack(
        [ftf[:, 7 - r: 7 - r + 8 * 15 + W] for r in range(8)], axis=1
    )
    tabv = jnp.stack(
        [tabr8[:, :, 8 * m: 8 * m + W] for m in range(15, -1, -1)], axis=1
    )
    tabv = tabv.reshape(H, 128, W)
    return _sc_expand_call(tabv)


# speedup vs baseline: 1.1431x; 1.1431x over previous
import functools

import jax
import jax.numpy as jnp
from jax import lax
from jax.experimental import pallas as pl
from jax.experimental.pallas import tpu as pltpu
from jax.experimental.pallas import tpu_sc as plsc

H = 16
S = 2048
NC = 2
NS = 16
NW = NC * NS
W = 15 * 128 + S
PHASES = 2
M_PER_TILE = 4


def _sc_expand_call(tabv):
    mesh = plsc.VectorSubcoreMesh(core_axis_name="c", subcore_axis_name="s")

    @functools.partial(
        pl.kernel,
        mesh=mesh,
        out_type=jax.ShapeDtypeStruct((H, S, S), jnp.float32),
        scratch_types=[
            pltpu.VMEM((8 * M_PER_TILE, W), jnp.float32),
            pltpu.SemaphoreType.DMA,
        ],
    )
    def sc_expand(tabv_hbm, out_hbm, vt, sem):
        cid = lax.axis_index("c")
        sid = lax.axis_index("s")
        w = sid * NC + cid
        hslot = w // 4
        mlo = M_PER_TILE * (w % 4)

        for phase in range(PHASES):
            h = phase * (H // PHASES) + hslot
            pltpu.sync_copy(tabv_hbm.at[h, pl.ds(8 * (12 - mlo), 32)], vt)

            def fire(a, carry):
                src = vt.at[:, pl.ds(128 * a, S)]
                dst = out_hbm.at[h, pl.ds(2016 - 8 * mlo - 128 * a, 32)]
                pltpu.make_async_copy(src, dst, sem).start()
                return carry

            lax.fori_loop(0, 16, fire, 0)

            def drain(a, carry):
                src = vt.at[:, pl.ds(0, S)]
                dst = out_hbm.at[h, pl.ds(2016 - 8 * mlo - 128 * a, 32)]
                pltpu.make_async_copy(src, dst, sem).wait()
                return carry

            lax.fori_loop(0, 16, drain, 0)

    return sc_expand(tabv)


def kernel(table, seq_len):
    del seq_len
    ftf = jnp.flip(table, axis=0).T
    tabr8 = jnp.stack(
        [ftf[:, 7 - r: 7 - r + 8 * 15 + W] for r in range(8)], axis=1
    )
    tabv = jnp.concatenate(
        [tabr8[:, :, 8 * m: 8 * m + W] for m in range(15, -1, -1)], axis=1
    )
    return _sc_expand_call(tabv)


# speedup vs baseline: 1.2023x; 1.0518x over previous
import functools

import jax
import jax.numpy as jnp
from jax import lax
from jax.experimental import pallas as pl
from jax.experimental.pallas import tpu as pltpu
from jax.experimental.pallas import tpu_sc as plsc

H = 16
S = 2048
NC = 2
NS = 16
NW = NC * NS
W = 15 * 128 + S
PHASES = 2
M_PER_TILE = 4


def _sc_expand_call(tabv):
    mesh = plsc.VectorSubcoreMesh(core_axis_name="c", subcore_axis_name="s")

    @functools.partial(
        pl.kernel,
        mesh=mesh,
        out_type=jax.ShapeDtypeStruct((H, S, S), jnp.float32),
        scratch_types=[
            pltpu.VMEM((16, W), jnp.float32),
            pltpu.VMEM((16, W), jnp.float32),
            pltpu.SemaphoreType.DMA,
            pltpu.SemaphoreType.DMA,
        ],
    )
    def sc_expand(tabv_hbm, out_hbm, vta, vtb, semf, sems):
        cid = lax.axis_index("c")
        sid = lax.axis_index("s")
        w = sid * NC + cid
        hslot = w // 4
        mlo = M_PER_TILE * (w % 4)

        def stage(u):
            h = (u // 2) * (H // PHASES) + hslot
            src = tabv_hbm.at[h, pl.ds(8 * (12 - mlo) + 16 * (u % 2), 16)]
            return pltpu.make_async_copy(src, vta if u % 2 == 0 else vtb, sems)

        def fires(u, start):
            vt = vta if u % 2 == 0 else vtb
            h = (u // 2) * (H // PHASES) + hslot
            base = 2016 - 8 * mlo + 16 * (u % 2)

            def body(a, carry):
                src = vt.at[:, pl.ds(128 * a, S)]
                dst = out_hbm.at[h, pl.ds(base - 128 * a, 16)]
                cp = pltpu.make_async_copy(src, dst, semf)
                cp.start() if start else cp.wait()
                return carry

            lax.fori_loop(0, 16, body, 0)

        st = stage(0)
        st.start()
        st.wait()
        for u in range(2 * PHASES):
            fires(u, start=True)
            if u + 1 < 2 * PHASES:
                stage(u + 1).start()
            fires(u, start=False)
            if u + 1 < 2 * PHASES:
                stage(u + 1).wait()

    return sc_expand(tabv)


def kernel(table, seq_len):
    del seq_len
    ftf_pad = jnp.flip(jnp.pad(table, ((0, 1), (0, 0))), axis=0).T
    tabr8 = jnp.stack(
        [ftf_pad[:, 8 - r: 8 - r + 8 * 15 + W] for r in range(8)], axis=1
    )
    tabv = jnp.concatenate(
        [tabr8[:, :, 8 * m: 8 * m + W] for m in range(15, -1, -1)], axis=1
    )
    return _sc_expand_call(tabv)
